# baseline (device time: 50802 ns/iter reference)
import jax
import jax.numpy as jnp
from jax import lax
from jax.experimental import pallas as pl
from jax.experimental.pallas import tpu as pltpu

N_DEV = 4
B, SQ, SKV, DH = 2, 256, 256, 64
HQ_LOCAL = 4
D_MODEL = 512
F_LOCAL = HQ_LOCAL * DH


def kernel(x, Wq, K_ext, V_ext, Wo):
    def body(x_ref, wq_ref, k_ref, v_ref, wo_ref, out_ref,
             comm_ref, send_sems, recv_sems):
        my = lax.axis_index("i")
        left = (my - 1) % N_DEV
        right = (my + 1) % N_DEV

        barrier_sem = pltpu.get_barrier_semaphore()
        for nbr in (left, right):
            pl.semaphore_signal(barrier_sem, inc=1, device_id=(nbr,),
                                device_id_type=pl.DeviceIdType.MESH)
        pl.semaphore_wait(barrier_sem, 2)

        qb = lax.broadcasted_iota(jnp.int32, (SQ, SKV), 0) // 64
        kb = lax.broadcasted_iota(jnp.int32, (SQ, SKV), 1) // 64
        mask = (qb == kb) | ((kb % 4) == (qb % 4))

        wq_l = wq_ref[:, pl.ds(my * F_LOCAL, F_LOCAL)]
        wo_l = wo_ref[pl.ds(my * F_LOCAL, F_LOCAL), :]

        for b in range(B):
            q_b = jnp.dot(x_ref[b], wq_l,
                          preferred_element_type=jnp.float32)
            ctx_heads = []
            for h in range(HQ_LOCAL):
                q_h = q_b[:, h * DH:(h + 1) * DH]
                k_h = k_ref[b, :, h, :]
                v_h = v_ref[b, :, h, :]
                s = jnp.dot(q_h, k_h.T,
                            preferred_element_type=jnp.float32) * 0.125
                s = jnp.where(mask, s, -1e9)
                m = jnp.max(s, axis=-1, keepdims=True)
                w = jnp.exp(s - m)
                w = w / jnp.sum(w, axis=-1, keepdims=True)
                ctx_heads.append(jnp.dot(w, v_h,
                                         preferred_element_type=jnp.float32))
            ctx_b = jnp.concatenate(ctx_heads, axis=-1)
            out_ref[b] = jnp.dot(ctx_b, wo_l,
                                 preferred_element_type=jnp.float32)

        comm_ref[0] = out_ref[...]

        for hop in range(N_DEV - 1):
            send_slot = hop % 2
            recv_slot = (hop + 1) % 2
            rdma = pltpu.make_async_remote_copy(
                src_ref=comm_ref.at[send_slot],
                dst_ref=comm_ref.at[recv_slot],
                send_sem=send_sems.at[send_slot],
                recv_sem=recv_sems.at[recv_slot],
                device_id=(right,),
                device_id_type=pl.DeviceIdType.MESH,
            )
            rdma.start()
            rdma.wait()
            out_ref[...] += comm_ref[recv_slot]

    return pl.pallas_call(
        body,
        out_shape=jax.ShapeDtypeStruct((B, SQ, D_MODEL), jnp.float32),
        in_specs=[pl.BlockSpec(memory_space=pltpu.VMEM)] * 5,
        out_specs=pl.BlockSpec(memory_space=pltpu.VMEM),
        scratch_shapes=[
            pltpu.VMEM((2, B, SQ, D_MODEL), jnp.float32),
            pltpu.SemaphoreType.DMA((2,)),
            pltpu.SemaphoreType.DMA((2,)),
        ],
        compiler_params=pltpu.CompilerParams(collective_id=0),
    )(x, Wq, K_ext, V_ext, Wo)


# device time: 25882 ns/iter; 1.9628x vs baseline; 1.9628x over previous
import jax
import jax.numpy as jnp
from jax import lax
from jax.experimental import pallas as pl
from jax.experimental.pallas import tpu as pltpu

N_DEV = 4
B, SQ, SKV, DH = 2, 256, 256, 64
HQ_LOCAL = 4
D_MODEL = 512
F_LOCAL = HQ_LOCAL * DH
ROWS = B * SQ
QROWS = ROWS // N_DEV


def kernel(x, Wq, K_ext, V_ext, Wo):
    x_flat = x.reshape(ROWS, D_MODEL)

    def body(x_ref, wq_ref, k_ref, v_ref, wo_ref, out_ref,
             stage_ref, rs_buf, rs_send, rs_recv, ag_send, ag_recv):
        my = lax.axis_index("i")

        barrier_sem = pltpu.get_barrier_semaphore()
        for k in range(1, N_DEV):
            pl.semaphore_signal(barrier_sem, inc=1,
                                device_id=((my + k) % N_DEV,),
                                device_id_type=pl.DeviceIdType.MESH)
        pl.semaphore_wait(barrier_sem, N_DEV - 1)

        wq_l = wq_ref[:, pl.ds(my * F_LOCAL, F_LOCAL)]
        wo_l = wo_ref[pl.ds(my * F_LOCAL, F_LOCAL), :]

        def compute_quarter(q):
            b = q // 2
            r0 = (q % 2) * QROWS
            xq = x_ref[pl.ds(q * QROWS, QROWS), :]
            qp = jnp.dot(xq, wq_l, preferred_element_type=jnp.float32)
            qi = r0 + lax.broadcasted_iota(jnp.int32, (QROWS, SKV), 0)
            ki = lax.broadcasted_iota(jnp.int32, (QROWS, SKV), 1)
            qb, kb = qi // 64, ki // 64
            mask = (qb == kb) | ((kb % 4) == (qb % 4))
            ctx_heads = []
            for h in range(HQ_LOCAL):
                q_h = qp[:, h * DH:(h + 1) * DH]
                k_h = k_ref[b, :, h, :]
                v_h = v_ref[b, :, h, :]
                s = jnp.dot(q_h, k_h.T,
                            preferred_element_type=jnp.float32) * 0.125
                s = jnp.where(mask, s, -1e9)
                m = jnp.max(s, axis=-1, keepdims=True)
                w = jnp.exp(s - m)
                w = w / jnp.sum(w, axis=-1, keepdims=True)
                ctx_heads.append(jnp.dot(w, v_h,
                                         preferred_element_type=jnp.float32))
            ctx = jnp.concatenate(ctx_heads, axis=-1)
            return jnp.dot(ctx, wo_l, preferred_element_type=jnp.float32)

        rs_rdmas = []
        for k in range(1, N_DEV):
            q = (my + k) % N_DEV
            stage_ref[k - 1] = compute_quarter(q)
            rdma = pltpu.make_async_remote_copy(
                src_ref=stage_ref.at[k - 1],
                dst_ref=rs_buf.at[k - 1],
                send_sem=rs_send.at[k - 1],
                recv_sem=rs_recv.at[k - 1],
                device_id=((my + k) % N_DEV,),
                device_id_type=pl.DeviceIdType.MESH,
            )
            rdma.start()
            rs_rdmas.append(rdma)

        stage_ref[N_DEV - 1] = compute_quarter(my)

        for rdma in rs_rdmas:
            rdma.wait_recv()
        red = (stage_ref[N_DEV - 1] + rs_buf[0] + rs_buf[1] + rs_buf[2])
        out_ref[pl.ds(my * QROWS, QROWS), :] = red

        ag_rdmas = []
        for k in range(1, N_DEV):
            rdma = pltpu.make_async_remote_copy(
                src_ref=out_ref.at[pl.ds(my * QROWS, QROWS), :],
                dst_ref=out_ref.at[pl.ds(my * QROWS, QROWS), :],
                send_sem=ag_send.at[k - 1],
                recv_sem=ag_recv.at[k - 1],
                device_id=((my + k) % N_DEV,),
                device_id_type=pl.DeviceIdType.MESH,
            )
            rdma.start()
            ag_rdmas.append(rdma)
        for rdma in ag_rdmas:
            rdma.wait_recv()
        for rdma in rs_rdmas + ag_rdmas:
            rdma.wait_send()

    out_flat = pl.pallas_call(
        body,
        out_shape=jax.ShapeDtypeStruct((ROWS, D_MODEL), jnp.float32),
        in_specs=[pl.BlockSpec(memory_space=pltpu.VMEM)] * 5,
        out_specs=pl.BlockSpec(memory_space=pltpu.VMEM),
        scratch_shapes=[
            pltpu.VMEM((N_DEV, QROWS, D_MODEL), jnp.float32),
            pltpu.VMEM((N_DEV - 1, QROWS, D_MODEL), jnp.float32),
            pltpu.SemaphoreType.DMA((N_DEV - 1,)),
            pltpu.SemaphoreType.DMA((N_DEV - 1,)),
            pltpu.SemaphoreType.DMA((N_DEV - 1,)),
            pltpu.SemaphoreType.DMA((N_DEV - 1,)),
        ],
        compiler_params=pltpu.CompilerParams(collective_id=0),
    )(x_flat, Wq, K_ext, V_ext, Wo)
    return out_flat.reshape(B, SQ, D_MODEL)
